# baseline (speedup 1.0000x reference)
import functools

import jax
import jax.numpy as jnp
from jax import lax
from jax.experimental import pallas as pl
from jax.experimental.pallas import tpu as pltpu
from jax.experimental.pallas import tpu_sc as plsc

NUM_EMBEDDINGS = 100000
EMBEDDING_DIM = 64
BATCH = 16384

NUM_CORES = 2
NUM_SUBCORES = 16
NUM_WORKERS = NUM_CORES * NUM_SUBCORES
COLS_PER_WORKER = EMBEDDING_DIM // NUM_WORKERS
QUARTER = BATCH // 4
L = 16


@functools.partial(
    pl.kernel,
    mesh=plsc.VectorSubcoreMesh(core_axis_name="c", subcore_axis_name="s"),
    out_type=jax.ShapeDtypeStruct((EMBEDDING_DIM, BATCH), jnp.float32),
    scratch_types=[
        pltpu.VMEM((NUM_EMBEDDINGS,), jnp.float32),
        pltpu.VMEM((BATCH,), jnp.int32),
        pltpu.VMEM((2, QUARTER), jnp.float32),
        pltpu.SemaphoreType.DMA,
        pltpu.SemaphoreType.DMA,
    ],
    compiler_params=pltpu.CompilerParams(needs_layout_passes=False),
)
def _sc_gather(tableT, idx_hbm, outT, col_v, idx_v, out_v, sem_col, sem_out):
    wid = lax.axis_index("s") * NUM_CORES + lax.axis_index("c")

    def start_col_copy(c):
        return [pltpu.async_copy(tableT.at[c], col_v, sem_col)]

    col0 = wid * COLS_PER_WORKER
    col_copies = start_col_copy(col0)
    pltpu.sync_copy(idx_hbm.at[pl.ds(0, BATCH)], idx_v)
    for d in col_copies:
        d.wait()

    pending = [None, None]
    col_copies = None
    for ci in range(COLS_PER_WORKER):
        c = col0 + ci
        if ci > 0:
            for d in col_copies:
                d.wait()
        for q in range(4):
            b = q % 2
            if pending[b] is not None:
                pending[b].wait()
                pending[b] = None

            @plsc.parallel_loop(0, QUARTER // L, unroll=16)
            def group_body(g):
                i16 = idx_v[pl.ds(q * QUARTER + g * L, L)]
                out_v[b, pl.ds(g * L, L)] = plsc.load_gather(col_v, [i16])

            d = pltpu.async_copy(
                out_v.at[b], outT.at[c, pl.ds(q * QUARTER, QUARTER)], sem_out
            )
            pending[b] = d
            if ci + 1 < COLS_PER_WORKER and q == 3:
                col_copies = start_col_copy(c + 1)
    pending[0].wait()
    pending[1].wait()


@jax.jit
def kernel(input, weight):
    outT = _sc_gather(weight.T, input.astype(jnp.int32))
    return outT.T


# speedup vs baseline: 1.0149x; 1.0149x over previous
import functools

import jax
import jax.numpy as jnp
from jax import lax
from jax.experimental import pallas as pl
from jax.experimental.pallas import tpu as pltpu
from jax.experimental.pallas import tpu_sc as plsc

NUM_EMBEDDINGS = 100000
EMBEDDING_DIM = 64
BATCH = 16384

NUM_CORES = 2
NUM_SUBCORES = 16
NUM_WORKERS = NUM_CORES * NUM_SUBCORES
COLS_PER_WORKER = EMBEDDING_DIM // NUM_WORKERS
QUARTER = BATCH // 4
L = 16


@functools.partial(
    pl.kernel,
    mesh=plsc.VectorSubcoreMesh(core_axis_name="c", subcore_axis_name="s"),
    out_type=jax.ShapeDtypeStruct((EMBEDDING_DIM, BATCH), jnp.float32),
    scratch_types=[
        pltpu.VMEM((NUM_EMBEDDINGS,), jnp.float32),
        pltpu.VMEM((BATCH,), jnp.int32),
        pltpu.VMEM((2, QUARTER), jnp.float32),
        pltpu.SemaphoreType.DMA,
        pltpu.SemaphoreType.DMA,
    ],
    compiler_params=pltpu.CompilerParams(
        needs_layout_passes=False,
        skip_device_barrier=True,
        disable_bounds_checks=True,
        disable_semaphore_checks=True,
    ),
)
def _sc_gather(tableT, idx_hbm, outT, col_v, idx_v, out_v, sem_col, sem_out):
    wid = lax.axis_index("s") * NUM_CORES + lax.axis_index("c")

    def start_col_copy(c):
        return [pltpu.async_copy(tableT.at[c], col_v, sem_col)]

    col0 = wid * COLS_PER_WORKER
    col_copies = start_col_copy(col0)
    pltpu.sync_copy(idx_hbm.at[pl.ds(0, BATCH)], idx_v)
    for d in col_copies:
        d.wait()

    pending = [None, None]
    col_copies = None
    for ci in range(COLS_PER_WORKER):
        c = col0 + ci
        if ci > 0:
            for d in col_copies:
                d.wait()
        for q in range(4):
            b = q % 2
            if pending[b] is not None:
                pending[b].wait()
                pending[b] = None

            @plsc.parallel_loop(0, QUARTER // L, unroll=8)
            def group_body(g):
                i16 = idx_v[pl.ds(q * QUARTER + g * L, L)]
                out_v[b, pl.ds(g * L, L)] = plsc.load_gather(col_v, [i16])

            d = pltpu.async_copy(
                out_v.at[b], outT.at[c, pl.ds(q * QUARTER, QUARTER)], sem_out
            )
            pending[b] = d
            if ci + 1 < COLS_PER_WORKER and q == 3:
                col_copies = start_col_copy(c + 1)
    pending[0].wait()
    pending[1].wait()


@jax.jit
def kernel(input, weight):
    outT = _sc_gather(weight.T, input.astype(jnp.int32))
    return outT.T
